# Initial kernel scaffold; baseline (speedup 1.0000x reference)
import jax
import jax.numpy as jnp
from jax.experimental import pallas as pl

N = 10000
K = 30
NP = 10240
BR = 128
KPAD = 32


def _knn_body(pv_ref, qr_ref, pt_ref, qc_ref, p_ref, idx_ref, sum_ref):
    pv = pv_ref[...]
    qr = qr_ref[...]
    pt = pt_ref[...]
    qc = qc_ref[...]
    g = pv[:, 0:1] * pt[0:1, :]
    g = g + pv[:, 1:2] * pt[1:2, :]
    g = g + pv[:, 2:3] * pt[2:3, :]
    d2 = (qr + qc) - 2.0 * g

    col = jax.lax.broadcasted_iota(jnp.float32, (BR, NP), 1)
    inf = jnp.float32(jnp.inf)
    cols = []
    for _ in range(K):
        m = jnp.min(d2, axis=1, keepdims=True)
        amin = jnp.min(jnp.where(d2 == m, col, jnp.float32(NP)),
                       axis=1, keepdims=True)
        cols.append(amin)
        d2 = jnp.where(col == amin, inf, d2)
    idx = jnp.concatenate(cols, axis=1).astype(jnp.int32)
    idx_ref[...] = jnp.concatenate(
        [idx, jnp.zeros((BR, KPAD - K), jnp.int32)], axis=1)
    sel = (d2 == inf).astype(jnp.float32)
    sum_ref[...] = jnp.dot(sel, p_ref[...],
                           preferred_element_type=jnp.float32)


def kernel(c, u, s, embedding1, embedding2):
    points = jnp.stack([c, u, s], axis=1)
    n = points.shape[0]
    mean = jnp.mean(points, axis=0, keepdims=True)
    pc = points - mean
    cov = (pc.T @ pc) / (n - 1)
    vi = jnp.linalg.inv(cov)
    pv = points @ vi
    q = jnp.einsum('ij,ij->i', pv, points)

    pad = NP - n
    f0 = jnp.zeros((pad,), jnp.float32)
    pv_pad = jnp.concatenate([pv, jnp.zeros((pad, 3), jnp.float32)], 0)
    qr = jnp.concatenate([q, f0], 0)[:, None]
    pt = jnp.concatenate([points.T, jnp.zeros((3, pad), jnp.float32)], 1)
    qc = jnp.concatenate([q, jnp.full((pad,), 1e30, jnp.float32)], 0)[None, :]
    p_pad = jnp.concatenate([points, jnp.zeros((pad, 3), jnp.float32)], 0)

    idx_out, sum_out = pl.pallas_call(
        _knn_body,
        grid=(NP // BR,),
        in_specs=[
            pl.BlockSpec((BR, 3), lambda i: (i, 0)),
            pl.BlockSpec((BR, 1), lambda i: (i, 0)),
            pl.BlockSpec((3, NP), lambda i: (0, 0)),
            pl.BlockSpec((1, NP), lambda i: (0, 0)),
            pl.BlockSpec((NP, 3), lambda i: (0, 0)),
        ],
        out_specs=[
            pl.BlockSpec((BR, KPAD), lambda i: (i, 0)),
            pl.BlockSpec((BR, 3), lambda i: (i, 0)),
        ],
        out_shape=[
            jax.ShapeDtypeStruct((NP, KPAD), jnp.int32),
            jax.ShapeDtypeStruct((NP, 3), jnp.float32),
        ],
    )(pv_pad, qr, pt, qc, p_pad)

    indices = idx_out[:N, :K]
    out = sum_out[:N] / jnp.float32(K)
    return out, indices


# speedup vs baseline: 4.4858x; 4.4858x over previous
import jax
import jax.numpy as jnp
from jax.experimental import pallas as pl

N = 10000
K = 30
NP = 10240
BR = 128
KPAD = 32


def _knn_body(pv_ref, qr_ref, pt_ref, qc_ref, p_ref, idx_ref, sum_ref):
    pv = pv_ref[...].astype(jnp.float32)
    qr = qr_ref[...]
    pt = pt_ref[...].astype(jnp.float32)
    qc = qc_ref[...]
    g = pv[:, 0:1] * pt[0:1, :]
    g = g + pv[:, 1:2] * pt[1:2, :]
    g = g + pv[:, 2:3] * pt[2:3, :]
    d2 = (qr + qc) - 2.0 * g

    col = jax.lax.broadcasted_iota(jnp.int32, (BR, NP), 1)
    inf = jnp.float32(jnp.inf)
    cols = []
    for _ in range(K):
        m = jnp.min(d2, axis=1, keepdims=True)
        amin = jnp.min(jnp.where(d2 == m, col, jnp.int32(NP)),
                       axis=1, keepdims=True)
        cols.append(amin)
        d2 = jnp.where(col == amin, inf, d2)
    idx = jnp.concatenate(cols, axis=1)
    idx_ref[...] = jnp.concatenate(
        [idx, jnp.zeros((BR, KPAD - K), jnp.int32)], axis=1)
    sel = (d2 == inf).astype(jnp.float32)
    sum_ref[...] = jnp.dot(sel, p_ref[...],
                           preferred_element_type=jnp.float32)


def kernel(c, u, s, embedding1, embedding2):
    points = jnp.stack([c, u, s], axis=1)
    n = points.shape[0]
    mean = jnp.mean(points, axis=0, keepdims=True)
    pc = points - mean
    cov = (pc.T @ pc) / (n - 1)
    vi = jnp.linalg.inv(cov)
    pv = points @ vi
    q = jnp.einsum('ij,ij->i', pv, points)

    pad = NP - n
    f0 = jnp.zeros((pad,), jnp.float32)
    pv_pad = jnp.concatenate(
        [pv.astype(jnp.bfloat16), jnp.zeros((pad, 3), jnp.bfloat16)], 0)
    qr = jnp.concatenate([q, f0], 0)[:, None]
    pt = jnp.concatenate(
        [points.T.astype(jnp.bfloat16), jnp.zeros((3, pad), jnp.bfloat16)], 1)
    qc = jnp.concatenate([q, jnp.full((pad,), 1e30, jnp.float32)], 0)[None, :]
    p_pad = jnp.concatenate([points, jnp.zeros((pad, 3), jnp.float32)], 0)

    idx_out, sum_out = pl.pallas_call(
        _knn_body,
        grid=(NP // BR,),
        in_specs=[
            pl.BlockSpec((BR, 3), lambda i: (i, 0)),
            pl.BlockSpec((BR, 1), lambda i: (i, 0)),
            pl.BlockSpec((3, NP), lambda i: (0, 0)),
            pl.BlockSpec((1, NP), lambda i: (0, 0)),
            pl.BlockSpec((NP, 3), lambda i: (0, 0)),
        ],
        out_specs=[
            pl.BlockSpec((BR, KPAD), lambda i: (i, 0)),
            pl.BlockSpec((BR, 3), lambda i: (i, 0)),
        ],
        out_shape=[
            jax.ShapeDtypeStruct((NP, KPAD), jnp.int32),
            jax.ShapeDtypeStruct((NP, 3), jnp.float32),
        ],
    )(pv_pad, qr, pt, qc, p_pad)

    indices = idx_out[:N, :K]
    out = sum_out[:N] / jnp.float32(K)
    return out, indices


# speedup vs baseline: 4.5854x; 1.0222x over previous
import jax
import jax.numpy as jnp
from jax.experimental import pallas as pl

N = 10000
K = 30
NP = 10112
BR = 128
KPAD = 32


def _knn_body(pv_ref, qr_ref, pt_ref, qc_ref, p_ref, idx_ref, sum_ref):
    pv = pv_ref[...].astype(jnp.float32)
    qr = qr_ref[...]
    pt = pt_ref[...].astype(jnp.float32)
    qc = qc_ref[...]
    g = pv[:, 0:1] * pt[0:1, :]
    g = g + pv[:, 1:2] * pt[1:2, :]
    g = g + pv[:, 2:3] * pt[2:3, :]
    d2 = (qr + qc) - 2.0 * g

    col = jax.lax.broadcasted_iota(jnp.int32, (BR, NP), 1)
    inf = jnp.float32(jnp.inf)
    cols = []
    for _ in range(K):
        m = jnp.min(d2, axis=1, keepdims=True)
        amin = jnp.min(jnp.where(d2 == m, col, jnp.int32(NP)),
                       axis=1, keepdims=True)
        cols.append(amin)
        d2 = jnp.where(col == amin, inf, d2)
    idx = jnp.concatenate(cols, axis=1)
    idx_ref[...] = jnp.concatenate(
        [idx, jnp.zeros((BR, KPAD - K), jnp.int32)], axis=1)
    sel = (d2 == inf).astype(jnp.float32)
    sum_ref[...] = jnp.dot(sel, p_ref[...],
                           preferred_element_type=jnp.float32)


def kernel(c, u, s, embedding1, embedding2):
    points = jnp.stack([c, u, s], axis=1)
    n = points.shape[0]
    mean = jnp.mean(points, axis=0, keepdims=True)
    pc = points - mean
    cov = (pc.T @ pc) / (n - 1)
    vi = jnp.linalg.inv(cov)
    pv = points @ vi
    q = jnp.einsum('ij,ij->i', pv, points)

    pad = NP - n
    f0 = jnp.zeros((pad,), jnp.float32)
    pv_pad = jnp.concatenate(
        [pv.astype(jnp.bfloat16), jnp.zeros((pad, 3), jnp.bfloat16)], 0)
    qr = jnp.concatenate([q, f0], 0)[:, None]
    pt = jnp.concatenate(
        [points.T.astype(jnp.bfloat16), jnp.zeros((3, pad), jnp.bfloat16)], 1)
    qc = jnp.concatenate([q, jnp.full((pad,), 1e30, jnp.float32)], 0)[None, :]
    p_pad = jnp.concatenate([points, jnp.zeros((pad, 3), jnp.float32)], 0)

    idx_out, sum_out = pl.pallas_call(
        _knn_body,
        grid=(NP // BR,),
        in_specs=[
            pl.BlockSpec((BR, 3), lambda i: (i, 0)),
            pl.BlockSpec((BR, 1), lambda i: (i, 0)),
            pl.BlockSpec((3, NP), lambda i: (0, 0)),
            pl.BlockSpec((1, NP), lambda i: (0, 0)),
            pl.BlockSpec((NP, 3), lambda i: (0, 0)),
        ],
        out_specs=[
            pl.BlockSpec((BR, KPAD), lambda i: (i, 0)),
            pl.BlockSpec((BR, 3), lambda i: (i, 0)),
        ],
        out_shape=[
            jax.ShapeDtypeStruct((NP, KPAD), jnp.int32),
            jax.ShapeDtypeStruct((NP, 3), jnp.float32),
        ],
    )(pv_pad, qr, pt, qc, p_pad)

    indices = idx_out[:N, :K]
    out = sum_out[:N] / jnp.float32(K)
    return out, indices


# speedup vs baseline: 4.8314x; 1.0536x over previous
import jax
import jax.numpy as jnp
from jax.experimental import pallas as pl

N = 10000
K = 30
NP = 10112
BR = 128
KPAD = 32


def _knn_body(pv_ref, qr_ref, pt_ref, qc_ref, p_ref, idx_ref, sum_ref):
    pv = pv_ref[...].astype(jnp.float32)
    qr = qr_ref[...]
    pt = pt_ref[...].astype(jnp.float32)
    qc = qc_ref[...]
    g = pv[:, 0:1] * pt[0:1, :]
    g = g + pv[:, 1:2] * pt[1:2, :]
    g = g + pv[:, 2:3] * pt[2:3, :]
    d2 = (qr + qc) - 2.0 * g

    col = jax.lax.broadcasted_iota(jnp.int32, (BR, NP), 1)
    inf = jnp.float32(jnp.inf)
    cols = []
    for _ in range(K):
        amin = jnp.argmin(d2, axis=1).astype(jnp.int32)[:, None]
        cols.append(amin)
        d2 = jnp.where(col == amin, inf, d2)
    idx = jnp.concatenate(cols, axis=1)
    idx_ref[...] = jnp.concatenate(
        [idx, jnp.zeros((BR, KPAD - K), jnp.int32)], axis=1)
    sel = (d2 == inf).astype(jnp.float32)
    sum_ref[...] = jnp.dot(sel, p_ref[...],
                           preferred_element_type=jnp.float32)


def kernel(c, u, s, embedding1, embedding2):
    points = jnp.stack([c, u, s], axis=1)
    n = points.shape[0]
    mean = jnp.mean(points, axis=0, keepdims=True)
    pc = points - mean
    cov = (pc.T @ pc) / (n - 1)
    vi = jnp.linalg.inv(cov)
    pv = points @ vi
    q = jnp.einsum('ij,ij->i', pv, points)

    pad = NP - n
    f0 = jnp.zeros((pad,), jnp.float32)
    pv_pad = jnp.concatenate(
        [pv.astype(jnp.bfloat16), jnp.zeros((pad, 3), jnp.bfloat16)], 0)
    qr = jnp.concatenate([q, f0], 0)[:, None]
    pt = jnp.concatenate(
        [points.T.astype(jnp.bfloat16), jnp.zeros((3, pad), jnp.bfloat16)], 1)
    qc = jnp.concatenate([q, jnp.full((pad,), 1e30, jnp.float32)], 0)[None, :]
    p_pad = jnp.concatenate([points, jnp.zeros((pad, 3), jnp.float32)], 0)

    idx_out, sum_out = pl.pallas_call(
        _knn_body,
        grid=(NP // BR,),
        in_specs=[
            pl.BlockSpec((BR, 3), lambda i: (i, 0)),
            pl.BlockSpec((BR, 1), lambda i: (i, 0)),
            pl.BlockSpec((3, NP), lambda i: (0, 0)),
            pl.BlockSpec((1, NP), lambda i: (0, 0)),
            pl.BlockSpec((NP, 3), lambda i: (0, 0)),
        ],
        out_specs=[
            pl.BlockSpec((BR, KPAD), lambda i: (i, 0)),
            pl.BlockSpec((BR, 3), lambda i: (i, 0)),
        ],
        out_shape=[
            jax.ShapeDtypeStruct((NP, KPAD), jnp.int32),
            jax.ShapeDtypeStruct((NP, 3), jnp.float32),
        ],
    )(pv_pad, qr, pt, qc, p_pad)

    indices = idx_out[:N, :K]
    out = sum_out[:N] / jnp.float32(K)
    return out, indices


# speedup vs baseline: 5.3295x; 1.1031x over previous
import jax
import jax.numpy as jnp
from jax.experimental import pallas as pl

N = 10000
K = 30
NP = 10112
BR = 128
KPAD = 32


def _knn_body(pv_ref, qr_ref, pt_ref, qc_ref, p_ref, idx_ref, sum_ref):
    pv = pv_ref[...].astype(jnp.float32)
    qr = qr_ref[...]
    pt = pt_ref[...].astype(jnp.float32)
    qc = qc_ref[...]
    g = pv[:, 0:1] * pt[0:1, :]
    g = g + pv[:, 1:2] * pt[1:2, :]
    g = g + pv[:, 2:3] * pt[2:3, :]
    d2 = (qr + qc) - 2.0 * g

    colf = jax.lax.broadcasted_iota(jnp.int32, (BR, NP), 1).astype(jnp.float32)
    npf = jnp.float32(NP)
    inf = jnp.float32(jnp.inf)
    cols = []
    for _ in range(K):
        m = jnp.min(d2, axis=1, keepdims=True)
        amin = jnp.min(jnp.where(d2 == m, colf, npf),
                       axis=1, keepdims=True)
        cols.append(amin)
        d2 = jnp.where(colf == amin, inf, d2)
    idxf = jnp.concatenate(cols, axis=1)
    idx = idxf.astype(jnp.int32)
    idx_ref[...] = jnp.concatenate(
        [idx, jnp.zeros((BR, KPAD - K), jnp.int32)], axis=1)
    sel = (d2 == inf).astype(jnp.float32)
    sum_ref[...] = jnp.dot(sel, p_ref[...],
                           preferred_element_type=jnp.float32)


def kernel(c, u, s, embedding1, embedding2):
    points = jnp.stack([c, u, s], axis=1)
    n = points.shape[0]
    mean = jnp.mean(points, axis=0, keepdims=True)
    pc = points - mean
    cov = (pc.T @ pc) / (n - 1)
    vi = jnp.linalg.inv(cov)
    pv = points @ vi
    q = jnp.einsum('ij,ij->i', pv, points)

    pad = NP - n
    f0 = jnp.zeros((pad,), jnp.float32)
    pv_pad = jnp.concatenate(
        [pv.astype(jnp.bfloat16), jnp.zeros((pad, 3), jnp.bfloat16)], 0)
    qr = jnp.concatenate([q, f0], 0)[:, None]
    pt = jnp.concatenate(
        [points.T.astype(jnp.bfloat16), jnp.zeros((3, pad), jnp.bfloat16)], 1)
    qc = jnp.concatenate([q, jnp.full((pad,), 1e30, jnp.float32)], 0)[None, :]
    p_pad = jnp.concatenate([points, jnp.zeros((pad, 3), jnp.float32)], 0)

    idx_out, sum_out = pl.pallas_call(
        _knn_body,
        grid=(NP // BR,),
        in_specs=[
            pl.BlockSpec((BR, 3), lambda i: (i, 0)),
            pl.BlockSpec((BR, 1), lambda i: (i, 0)),
            pl.BlockSpec((3, NP), lambda i: (0, 0)),
            pl.BlockSpec((1, NP), lambda i: (0, 0)),
            pl.BlockSpec((NP, 3), lambda i: (0, 0)),
        ],
        out_specs=[
            pl.BlockSpec((BR, KPAD), lambda i: (i, 0)),
            pl.BlockSpec((BR, 3), lambda i: (i, 0)),
        ],
        out_shape=[
            jax.ShapeDtypeStruct((NP, KPAD), jnp.int32),
            jax.ShapeDtypeStruct((NP, 3), jnp.float32),
        ],
    )(pv_pad, qr, pt, qc, p_pad)

    indices = idx_out[:N, :K]
    out = sum_out[:N] / jnp.float32(K)
    return out, indices


# speedup vs baseline: 5.3320x; 1.0005x over previous
import jax
import jax.numpy as jnp
from jax.experimental import pallas as pl

N = 10000
K = 30
NP = 10112
BR = 128
NH = NP // 2
KPAD = 32


def _knn_body(pv_ref, qr_ref, pt_ref, qc_ref, p_ref, idx_ref, sum_ref):
    pv = pv_ref[...].astype(jnp.float32)
    qr = qr_ref[...]
    pt = pt_ref[...].astype(jnp.float32)
    qc = qc_ref[...]
    g = pv[:, 0:1] * pt[0:1, :]
    g = g + pv[:, 1:2] * pt[1:2, :]
    g = g + pv[:, 2:3] * pt[2:3, :]
    d2 = (qr + qc) - 2.0 * g

    a = d2[:, :NH]
    b = d2[:, NH:]
    ca = jax.lax.broadcasted_iota(jnp.int32, (BR, NH), 1).astype(jnp.float32)
    cb = ca + jnp.float32(NH)
    wb = b < a
    lo = jnp.where(wb, b, a)
    hi = jnp.where(wb, a, b)
    cl = jnp.where(wb, cb, ca)
    ch = jnp.where(wb, ca, cb)

    npf = jnp.float32(NP)
    inf = jnp.float32(jnp.inf)
    cols = []
    for _ in range(K):
        m = jnp.min(lo, axis=1, keepdims=True)
        amin = jnp.min(jnp.where(lo == m, cl, npf),
                       axis=1, keepdims=True)
        cols.append(amin)
        eqs = cl == amin
        lo = jnp.where(eqs, hi, lo)
        cl = jnp.where(eqs, ch, cl)
        hi = jnp.where(eqs, inf, hi)
    idxf = jnp.concatenate(cols, axis=1)
    idx = idxf.astype(jnp.int32)
    idx_ref[...] = jnp.concatenate(
        [idx, jnp.zeros((BR, KPAD - K), jnp.int32)], axis=1)
    lof = lo != inf
    hif = hi != inf
    one = jnp.float32(1.0)
    zero = jnp.float32(0.0)
    pres_a = ((cl == ca) & lof) | ((ch == ca) & hif)
    pres_b = ((cl == cb) & lof) | ((ch == cb) & hif)
    sel = jnp.concatenate([jnp.where(pres_a, zero, one),
                           jnp.where(pres_b, zero, one)], axis=1)
    sum_ref[...] = jnp.dot(sel, p_ref[...],
                           preferred_element_type=jnp.float32)


def kernel(c, u, s, embedding1, embedding2):
    points = jnp.stack([c, u, s], axis=1)
    n = points.shape[0]
    mean = jnp.mean(points, axis=0, keepdims=True)
    pc = points - mean
    cov = (pc.T @ pc) / (n - 1)
    vi = jnp.linalg.inv(cov)
    pv = points @ vi
    q = jnp.einsum('ij,ij->i', pv, points)

    pad = NP - n
    f0 = jnp.zeros((pad,), jnp.float32)
    pv_pad = jnp.concatenate(
        [pv.astype(jnp.bfloat16), jnp.zeros((pad, 3), jnp.bfloat16)], 0)
    qr = jnp.concatenate([q, f0], 0)[:, None]
    pt = jnp.concatenate(
        [points.T.astype(jnp.bfloat16), jnp.zeros((3, pad), jnp.bfloat16)], 1)
    qc = jnp.concatenate([q, jnp.full((pad,), 1e30, jnp.float32)], 0)[None, :]
    p_pad = jnp.concatenate([points, jnp.zeros((pad, 3), jnp.float32)], 0)

    idx_out, sum_out = pl.pallas_call(
        _knn_body,
        grid=(NP // BR,),
        in_specs=[
            pl.BlockSpec((BR, 3), lambda i: (i, 0)),
            pl.BlockSpec((BR, 1), lambda i: (i, 0)),
            pl.BlockSpec((3, NP), lambda i: (0, 0)),
            pl.BlockSpec((1, NP), lambda i: (0, 0)),
            pl.BlockSpec((NP, 3), lambda i: (0, 0)),
        ],
        out_specs=[
            pl.BlockSpec((BR, KPAD), lambda i: (i, 0)),
            pl.BlockSpec((BR, 3), lambda i: (i, 0)),
        ],
        out_shape=[
            jax.ShapeDtypeStruct((NP, KPAD), jnp.int32),
            jax.ShapeDtypeStruct((NP, 3), jnp.float32),
        ],
    )(pv_pad, qr, pt, qc, p_pad)

    indices = idx_out[:N, :K]
    out = sum_out[:N] / jnp.float32(K)
    return out, indices


# speedup vs baseline: 6.0626x; 1.1370x over previous
import jax
import jax.numpy as jnp
from jax.experimental import pallas as pl

N = 10000
K = 30
NP = 10112
BR = 256
NR = 10240
NH = NP // 2
KPAD = 32


def _knn_body(pv_ref, qr_ref, pt_ref, qc_ref, p_ref, idx_ref, sum_ref):
    pv = pv_ref[...].astype(jnp.float32)
    qr = qr_ref[...]
    pt = pt_ref[...].astype(jnp.float32)
    qc = qc_ref[...]
    g = pv[:, 0:1] * pt[0:1, :]
    g = g + pv[:, 1:2] * pt[1:2, :]
    g = g + pv[:, 2:3] * pt[2:3, :]
    d2 = (qr + qc) - 2.0 * g

    a = d2[:, :NH]
    b = d2[:, NH:]
    ca = jax.lax.broadcasted_iota(jnp.int32, (BR, NH), 1).astype(jnp.float32)
    cb = ca + jnp.float32(NH)
    wb = b < a
    lo = jnp.where(wb, b, a)
    hi = jnp.where(wb, a, b)
    cl = jnp.where(wb, cb, ca)
    ch = jnp.where(wb, ca, cb)

    npf = jnp.float32(NP)
    inf = jnp.float32(jnp.inf)
    cols = []
    for _ in range(K):
        m = jnp.min(lo, axis=1, keepdims=True)
        amin = jnp.min(jnp.where(lo == m, cl, npf),
                       axis=1, keepdims=True)
        cols.append(amin)
        eqs = cl == amin
        lo = jnp.where(eqs, hi, lo)
        cl = jnp.where(eqs, ch, cl)
        hi = jnp.where(eqs, inf, hi)
    idxf = jnp.concatenate(cols, axis=1)
    idx = idxf.astype(jnp.int32)
    idx_ref[...] = jnp.concatenate(
        [idx, jnp.zeros((BR, KPAD - K), jnp.int32)], axis=1)
    lof = lo != inf
    hif = hi != inf
    one = jnp.float32(1.0)
    zero = jnp.float32(0.0)
    pres_a = ((cl == ca) & lof) | ((ch == ca) & hif)
    pres_b = ((cl == cb) & lof) | ((ch == cb) & hif)
    sel = jnp.concatenate([jnp.where(pres_a, zero, one),
                           jnp.where(pres_b, zero, one)], axis=1)
    sum_ref[...] = jnp.dot(sel, p_ref[...],
                           preferred_element_type=jnp.float32)


def kernel(c, u, s, embedding1, embedding2):
    points = jnp.stack([c, u, s], axis=1)
    n = points.shape[0]
    mean = jnp.mean(points, axis=0, keepdims=True)
    pc = points - mean
    cov = (pc.T @ pc) / (n - 1)
    vi = jnp.linalg.inv(cov)
    pv = points @ vi
    q = jnp.einsum('ij,ij->i', pv, points)

    padr = NR - n
    padc = NP - n
    pv_pad = jnp.concatenate(
        [pv.astype(jnp.bfloat16), jnp.zeros((padr, 3), jnp.bfloat16)], 0)
    qr = jnp.concatenate(
        [q, jnp.zeros((padr,), jnp.float32)], 0)[:, None]
    pt = jnp.concatenate(
        [points.T.astype(jnp.bfloat16), jnp.zeros((3, padc), jnp.bfloat16)], 1)
    qc = jnp.concatenate([q, jnp.full((padc,), 1e30, jnp.float32)], 0)[None, :]
    p_pad = jnp.concatenate([points, jnp.zeros((padc, 3), jnp.float32)], 0)

    idx_out, sum_out = pl.pallas_call(
        _knn_body,
        grid=(NR // BR,),
        in_specs=[
            pl.BlockSpec((BR, 3), lambda i: (i, 0)),
            pl.BlockSpec((BR, 1), lambda i: (i, 0)),
            pl.BlockSpec((3, NP), lambda i: (0, 0)),
            pl.BlockSpec((1, NP), lambda i: (0, 0)),
            pl.BlockSpec((NP, 3), lambda i: (0, 0)),
        ],
        out_specs=[
            pl.BlockSpec((BR, KPAD), lambda i: (i, 0)),
            pl.BlockSpec((BR, 3), lambda i: (i, 0)),
        ],
        out_shape=[
            jax.ShapeDtypeStruct((NR, KPAD), jnp.int32),
            jax.ShapeDtypeStruct((NR, 3), jnp.float32),
        ],
    )(pv_pad, qr, pt, qc, p_pad)

    indices = idx_out[:N, :K]
    out = sum_out[:N] / jnp.float32(K)
    return out, indices


# speedup vs baseline: 6.5366x; 1.0782x over previous
import jax
import jax.numpy as jnp
from jax.experimental import pallas as pl

N = 10000
K = 30
NP = 10112
BR = 512
NR = 10240
NH = NP // 2
KPAD = 32


def _knn_body(pv_ref, qr_ref, pt_ref, qc_ref, p_ref, idx_ref, sum_ref):
    pv = pv_ref[...].astype(jnp.float32)
    qr = qr_ref[...]
    pt = pt_ref[...].astype(jnp.float32)
    qc = qc_ref[...]
    g = pv[:, 0:1] * pt[0:1, :]
    g = g + pv[:, 1:2] * pt[1:2, :]
    g = g + pv[:, 2:3] * pt[2:3, :]
    d2 = (qr + qc) - 2.0 * g

    a = d2[:, :NH]
    b = d2[:, NH:]
    ca = jax.lax.broadcasted_iota(jnp.int32, (BR, NH), 1).astype(jnp.float32)
    cb = ca + jnp.float32(NH)
    wb = b < a
    lo = jnp.where(wb, b, a)
    hi = jnp.where(wb, a, b)
    cl = jnp.where(wb, cb, ca)
    ch = jnp.where(wb, ca, cb)

    npf = jnp.float32(NP)
    inf = jnp.float32(jnp.inf)
    cols = []
    for _ in range(K):
        m = jnp.min(lo, axis=1, keepdims=True)
        amin = jnp.min(jnp.where(lo == m, cl, npf),
                       axis=1, keepdims=True)
        cols.append(amin)
        eqs = cl == amin
        lo = jnp.where(eqs, hi, lo)
        cl = jnp.where(eqs, ch, cl)
        hi = jnp.where(eqs, inf, hi)
    idxf = jnp.concatenate(cols, axis=1)
    idx = idxf.astype(jnp.int32)
    idx_ref[...] = jnp.concatenate(
        [idx, jnp.zeros((BR, KPAD - K), jnp.int32)], axis=1)
    lof = lo != inf
    hif = hi != inf
    one = jnp.float32(1.0)
    zero = jnp.float32(0.0)
    pres_a = ((cl == ca) & lof) | ((ch == ca) & hif)
    pres_b = ((cl == cb) & lof) | ((ch == cb) & hif)
    sel = jnp.concatenate([jnp.where(pres_a, zero, one),
                           jnp.where(pres_b, zero, one)], axis=1)
    sum_ref[...] = jnp.dot(sel, p_ref[...],
                           preferred_element_type=jnp.float32)


def kernel(c, u, s, embedding1, embedding2):
    points = jnp.stack([c, u, s], axis=1)
    n = points.shape[0]
    mean = jnp.mean(points, axis=0, keepdims=True)
    pc = points - mean
    cov = (pc.T @ pc) / (n - 1)
    vi = jnp.linalg.inv(cov)
    pv = points @ vi
    q = jnp.einsum('ij,ij->i', pv, points)

    padr = NR - n
    padc = NP - n
    pv_pad = jnp.concatenate(
        [pv.astype(jnp.bfloat16), jnp.zeros((padr, 3), jnp.bfloat16)], 0)
    qr = jnp.concatenate(
        [q, jnp.zeros((padr,), jnp.float32)], 0)[:, None]
    pt = jnp.concatenate(
        [points.T.astype(jnp.bfloat16), jnp.zeros((3, padc), jnp.bfloat16)], 1)
    qc = jnp.concatenate([q, jnp.full((padc,), 1e30, jnp.float32)], 0)[None, :]
    p_pad = jnp.concatenate([points, jnp.zeros((padc, 3), jnp.float32)], 0)

    idx_out, sum_out = pl.pallas_call(
        _knn_body,
        grid=(NR // BR,),
        in_specs=[
            pl.BlockSpec((BR, 3), lambda i: (i, 0)),
            pl.BlockSpec((BR, 1), lambda i: (i, 0)),
            pl.BlockSpec((3, NP), lambda i: (0, 0)),
            pl.BlockSpec((1, NP), lambda i: (0, 0)),
            pl.BlockSpec((NP, 3), lambda i: (0, 0)),
        ],
        out_specs=[
            pl.BlockSpec((BR, KPAD), lambda i: (i, 0)),
            pl.BlockSpec((BR, 3), lambda i: (i, 0)),
        ],
        out_shape=[
            jax.ShapeDtypeStruct((NR, KPAD), jnp.int32),
            jax.ShapeDtypeStruct((NR, 3), jnp.float32),
        ],
    )(pv_pad, qr, pt, qc, p_pad)

    indices = idx_out[:N, :K]
    out = sum_out[:N] / jnp.float32(K)
    return out, indices
